# baseline (speedup 1.0000x reference)
import functools

import jax
import jax.numpy as jnp
from jax import lax
from jax.experimental import pallas as pl
from jax.experimental.pallas import tpu as pltpu
from jax.experimental.pallas import tpu_sc as plsc

_NC, _NS = 2, 16
_NW = _NC * _NS
_D = 1024
_CB = 32
_CHUNK = 512
_N_SC = 8192


def _sc_part(xf, tz, n_sc):
    bpw = n_sc // _NW
    chunks = bpw // _CB
    nv = 3
    trep = jnp.tile(tz[None], (_NW, 1, 1)).reshape(_NW * nv, _D)
    xadj = xf[:n_sc] + nv * (jnp.arange(n_sc, dtype=jnp.int32) // bpw)

    mesh = plsc.VectorSubcoreMesh(core_axis_name="c", subcore_axis_name="s")

    @functools.partial(
        pl.kernel,
        out_type=jax.ShapeDtypeStruct((n_sc, _D), jnp.float32),
        mesh=mesh,
        scratch_types=[
            pltpu.VMEM((bpw,), jnp.int32),
            pltpu.VMEM((_CB, _D), jnp.float32),
            pltpu.VMEM((_CB, _D), jnp.float32),
            pltpu.SemaphoreType.DMA,
            pltpu.SemaphoreType.DMA,
            pltpu.SemaphoreType.DMA,
            pltpu.SemaphoreType.DMA,
        ],
    )
    def sc_emb(x_hbm, t_hbm, out_hbm, idx_v, buf0, buf1, gs0, gs1, ps0, ps1):
        wid = lax.axis_index("s") * _NC + lax.axis_index("c")
        base = wid * bpw
        pltpu.sync_copy(x_hbm.at[pl.ds(base, bpw)], idx_v)
        bufs = (buf0, buf1)
        gsems = (gs0, gs1)
        psems = (ps0, ps1)

        def start_gather(c, k):
            pltpu.async_copy(t_hbm.at[idx_v.at[pl.ds(c * _CB, _CB)]],
                             bufs[k], gsems[k])

        def wait_gather(k):
            pltpu.make_async_copy(t_hbm.at[idx_v.at[pl.ds(0, _CB)]],
                                  bufs[k], gsems[k]).wait()

        def start_put(c, k):
            pltpu.async_copy(bufs[k], out_hbm.at[pl.ds(base + c * _CB, _CB)],
                             psems[k])

        def wait_put(k):
            pltpu.make_async_copy(bufs[k], out_hbm.at[pl.ds(base, _CB)],
                                  psems[k]).wait()

        start_gather(0, 0)

        @pl.loop(0, chunks, step=2)
        def pair(g):
            wait_gather(0)

            @pl.when(g > 0)
            def _():
                wait_put(1)

            start_gather(g + 1, 1)
            start_put(g, 0)
            wait_gather(1)
            wait_put(0)

            @pl.when(g < chunks - 2)
            def _():
                start_gather(g + 2, 0)

            start_put(g + 1, 1)

        wait_put(1)

    return sc_emb(xadj, trep)


def _tc_body(x_ref, t_ref, o_ref):
    xc = x_ref[0, 0, :][:, None]
    r1 = t_ref[1, :][None, :]
    r2 = t_ref[2, :][None, :]
    w1 = (xc == 1).astype(jnp.float32)
    w2 = (xc == 2).astype(jnp.float32)
    o_ref[...] = w1 * r1 + w2 * r2


def _tc_part(xs, table, n_tc):
    grid = n_tc // _CHUNK
    x_r = xs.reshape(grid, 1, _CHUNK)
    return pl.pallas_call(
        _tc_body,
        grid=(grid,),
        in_specs=[
            pl.BlockSpec((1, 1, _CHUNK), lambda i: (i, 0, 0)),
            pl.BlockSpec((3, _D), lambda i: (0, 0)),
        ],
        out_specs=pl.BlockSpec((_CHUNK, _D), lambda i: (i, 0)),
        out_shape=jax.ShapeDtypeStruct((n_tc, _D), jnp.float32),
    )(x_r, table)


def kernel(x, table):
    b, s = x.shape
    n = b * s
    xf = x.reshape(n).astype(jnp.int32)
    tz = table.at[0].set(0.0)
    o_sc = _sc_part(xf, tz, _N_SC)
    n_tc = n - _N_SC
    o_tc1 = _tc_part(xf[_N_SC:_N_SC + n_tc // 4], table, n_tc // 4)
    o_tc2 = _tc_part(xf[_N_SC + n_tc // 4:], table, n_tc - n_tc // 4)
    return (o_sc, o_tc1, o_tc2)


# speedup vs baseline: 2.5587x; 2.5587x over previous
import jax
import jax.numpy as jnp
from jax.experimental import pallas as pl

_HIDDEN = 1024
_NUM_EMB = 3
_CHUNK = 1024


def _emb_body(x_ref, t_ref, o_ref):
    xc = x_ref[0, 0, :][:, None]
    r1 = t_ref[1, :][None, :]
    r2 = t_ref[2, :][None, :]
    zero = jnp.zeros((), jnp.float32)
    o_ref[...] = jnp.where(xc == 1, r1, jnp.where(xc == 2, r2, zero))


def kernel(x, table):
    b, s = x.shape
    n = b * s
    grid = n // _CHUNK
    x_r = x.reshape(grid, 1, _CHUNK).astype(jnp.int32)
    out = pl.pallas_call(
        _emb_body,
        grid=(grid,),
        in_specs=[
            pl.BlockSpec((1, 1, _CHUNK), lambda i: (i, 0, 0)),
            pl.BlockSpec((_NUM_EMB, _HIDDEN), lambda i: (0, 0)),
        ],
        out_specs=pl.BlockSpec((_CHUNK, _HIDDEN), lambda i: (i, 0)),
        out_shape=jax.ShapeDtypeStruct((n, _HIDDEN), jnp.float32),
    )(x_r, table)
    return out.reshape(b, s, _HIDDEN)
